# baseline (speedup 1.0000x reference)
import jax
import jax.numpy as jnp
from jax import lax
from jax.experimental import pallas as pl
from jax.experimental.pallas import tpu as pltpu
from jax.experimental.pallas import tpu_sc as plsc

N = 320000
D = 128
S = 10000
NW = 32
SPW = 320
S_PAD = NW * SPW
C = 128
SUB = 16
BCH = SUB * C
RPW = SPW + 1
CNT_PAD = ((SPW + 15) // 16 + 1) * 16


def _pool_kernel(x_hbm, b_hbm, bounds_hbm, out_hbm,
                 xbuf, bbuf, accl, cnt, bnds, sx0, sx1):
    cid = lax.axis_index("c")
    sid = lax.axis_index("s")
    w = sid * 2 + cid
    seg_lo = w * SPW

    sx = (sx0, sx1)

    pltpu.sync_copy(bounds_hbm, bnds)
    bv0 = bnds[pl.ds(w, 16)]
    lo = bv0[0]
    hi = bv0[1]

    zeros16 = jnp.zeros((16,), jnp.float32)

    def zrow(i, carry):
        for j in range(8):
            accl[i, pl.ds(j * 16, 16)] = zeros16
        return carry
    lax.fori_loop(0, RPW, zrow, 0)
    for j in range(CNT_PAD // 16):
        cnt[pl.ds(j * 16, 16)] = zeros16

    lo_al = lo & jnp.int32(~7)
    nchunks = (hi - lo_al + C - 1) // C

    def xload(k, b):
        r_eff = pl.multiple_of(jnp.minimum(lo_al + k * C, N - C), 8)
        pltpu.async_copy(x_hbm.at[pl.ds(r_eff, C)], xbuf.at[b], sx[b])

    def xwait(b):
        pltpu.make_async_copy(x_hbm.at[pl.ds(0, C)], xbuf.at[b],
                              sx[b]).wait()

    @pl.when(nchunks > 0)
    def _():
        xload(0, 0)

    def pair(p, carry):
        for b in range(2):
            k = 2 * p + b

            @pl.when(k < nchunks)
            def _():
                s_sup = k // SUB
                rb_eff = pl.multiple_of(
                    jnp.minimum(lo_al + s_sup * BCH, N - BCH), 8)

                @pl.when(lax.rem(k, SUB) == 0)
                def _():
                    pltpu.sync_copy(b_hbm.at[pl.ds(rb_eff, BCH)], bbuf)

                @pl.when(k + 1 < nchunks)
                def _():
                    xload(k + 1, 1 - b)

                r = lo_al + k * C
                r_eff = pl.multiple_of(jnp.minimum(r, N - C), 8)
                off = r_eff - rb_eff
                vlo = jnp.maximum(r, lo)

                xwait(b)

                def rows16(j, c2):
                    bv = bbuf[pl.ds(off + j * 16, 16)]
                    g = r_eff + j * 16 + lax.iota(jnp.int32, 16)
                    valid = (g >= vlo) & (g < hi)
                    loc = jnp.where(valid, bv - seg_lo, SPW)
                    ones = jnp.where(valid, 1.0, 0.0).astype(jnp.float32)
                    plsc.addupdate_scatter(cnt, [loc], ones)
                    ls = [loc[i] for i in range(16)]
                    for col in range(8):
                        for i in range(16):
                            xv = xbuf[b, j * 16 + i, pl.ds(col * 16, 16)]
                            plsc.addupdate(
                                accl.at[ls[i], pl.ds(col * 16, 16)], xv)
                    return c2
                lax.fori_loop(0, C // 16, rows16, 0)
        return carry
    lax.fori_loop(0, (nchunks + 1) // 2, pair, 0)

    def div_row(s, carry):
        cv = cnt[pl.ds(s, 16)]
        inv = (jnp.ones((16,), jnp.float32) / jnp.maximum(cv, 1.0))[0]
        for j in range(8):
            accl[s, pl.ds(j * 16, 16)] = accl[s, pl.ds(j * 16, 16)] * inv
        return carry
    lax.fori_loop(0, SPW, div_row, 0)

    pltpu.sync_copy(accl.at[pl.ds(0, SPW)], out_hbm.at[pl.ds(seg_lo, SPW)])


def kernel(x, batch):
    b32 = batch.astype(jnp.int32)
    edges = jnp.arange(NW + 1, dtype=jnp.int32) * SPW
    bounds = jnp.searchsorted(b32, edges, side="left").astype(jnp.int32)
    bounds = jnp.concatenate([bounds, jnp.zeros((15,), jnp.int32)])

    mesh = plsc.VectorSubcoreMesh(core_axis_name="c", subcore_axis_name="s")
    out = pl.kernel(
        _pool_kernel,
        mesh=mesh,
        compiler_params=pltpu.CompilerParams(needs_layout_passes=False),
        out_type=jax.ShapeDtypeStruct((S_PAD, D), jnp.float32),
        scratch_types=[
            pltpu.VMEM((2, C, D), jnp.float32),
            pltpu.VMEM((BCH,), jnp.int32),
            pltpu.VMEM((RPW, D), jnp.float32),
            pltpu.VMEM((CNT_PAD,), jnp.float32),
            pltpu.VMEM((48,), jnp.int32),
            pltpu.SemaphoreType.DMA,
            pltpu.SemaphoreType.DMA,
        ],
    )(x, b32, bounds)
    return out[:S]


# speedup vs baseline: 2.1291x; 2.1291x over previous
import jax
import jax.numpy as jnp
from jax import lax
from jax.experimental import pallas as pl
from jax.experimental.pallas import tpu as pltpu
from jax.experimental.pallas import tpu_sc as plsc

N = 320000
D = 128
S = 10000
NW = 32
SPW = 320
S_PAD = NW * SPW
C = 128
SUB = 16
BCH = SUB * C
RPW = SPW + 8
CNT_PAD = ((SPW + 15) // 16 + 1) * 16


NB = 4


def _pool_kernel(x_hbm, b_hbm, bounds_hbm, out_hbm,
                 xbuf, bbuf, idxb, cnt, bnds, acc_sh,
                 sx0, sx1, sx2, sx3, ss0, ss1, ss2, ss3):
    cid = lax.axis_index("c")
    sid = lax.axis_index("s")
    w = sid * 2 + cid
    seg_lo = w * SPW
    base = sid * RPW

    sx = (sx0, sx1, sx2, sx3)
    ss = (ss0, ss1, ss2, ss3)

    pltpu.sync_copy(bounds_hbm, bnds)
    bv0 = bnds[pl.ds(w, 16)]
    lo = bv0[0]
    hi = bv0[1]

    zeros16 = jnp.zeros((16,), jnp.float32)

    def zrow(i, carry):
        for j in range(8):
            xbuf[0, i, pl.ds(j * 16, 16)] = zeros16
        return carry
    lax.fori_loop(0, C, zrow, 0)
    pltpu.sync_copy(xbuf.at[0], acc_sh.at[pl.ds(base, C)])
    pltpu.sync_copy(xbuf.at[0], acc_sh.at[pl.ds(base + C, C)])
    pltpu.sync_copy(xbuf.at[0].at[pl.ds(0, RPW - 2 * C)],
                    acc_sh.at[pl.ds(base + 2 * C, RPW - 2 * C)])
    for j in range(CNT_PAD // 16):
        cnt[pl.ds(j * 16, 16)] = zeros16

    lo_al = lo & jnp.int32(~7)
    nchunks = (hi - lo_al + C - 1) // C

    def xload(k, b):
        r_eff = pl.multiple_of(jnp.minimum(lo_al + k * C, N - C), 8)
        pltpu.async_copy(x_hbm.at[pl.ds(r_eff, C)], xbuf.at[b], sx[b])

    def xwait(b):
        pltpu.make_async_copy(x_hbm.at[pl.ds(0, C)], xbuf.at[b],
                              sx[b]).wait()

    def scat_wait(b):
        pltpu.make_async_copy(xbuf.at[b], acc_sh.at[idxb.at[b]],
                              ss[b]).wait()

    @pl.when(nchunks > 0)
    def _():
        xload(0, 0)

    def pair(p, carry):
        for b in range(NB):
            k = NB * p + b

            @pl.when(k < nchunks)
            def _():
                s_sup = k // SUB
                rb_eff = pl.multiple_of(
                    jnp.minimum(lo_al + s_sup * BCH, N - BCH), 8)

                if b == 0:
                    @pl.when(lax.rem(k, SUB) == 0)
                    def _():
                        pltpu.sync_copy(b_hbm.at[pl.ds(rb_eff, BCH)], bbuf)

                @pl.when(k >= NB - 1)
                def _():
                    scat_wait((b + 1) % NB)

                @pl.when(k + 1 < nchunks)
                def _():
                    xload(k + 1, (b + 1) % NB)

                r = lo_al + k * C
                r_eff = pl.multiple_of(jnp.minimum(r, N - C), 8)
                off = r_eff - rb_eff
                vlo = jnp.maximum(r, lo)
                for j in range(C // 16):
                    bv = bbuf[pl.ds(off + j * 16, 16)]
                    g = r_eff + j * 16 + lax.iota(jnp.int32, 16)
                    valid = (g >= vlo) & (g < hi)
                    loc = jnp.where(valid, bv - seg_lo, SPW)
                    idxb[b, pl.ds(j * 16, 16)] = base + loc
                    ones = jnp.where(valid, 1.0, 0.0).astype(jnp.float32)
                    plsc.addupdate_scatter(cnt, [loc], ones)

                xwait(b)
                pltpu.async_copy(xbuf.at[b], acc_sh.at[idxb.at[b]], ss[b],
                                 add=True)
        return carry
    lax.fori_loop(0, (nchunks + NB - 1) // NB, pair, 0)

    for b in range(NB):
        cond = jnp.bool_(False)
        for t in range(1, NB):
            cond = cond | ((nchunks >= t) & (lax.rem(nchunks - t, NB) == b))

        @pl.when(cond)
        def _(b=b):
            scat_wait(b)

    W = 64
    blk = xbuf.at[0].at[pl.ds(0, W)]
    for t in range(SPW // W):
        pltpu.sync_copy(acc_sh.at[pl.ds(base + t * W, W)], blk)

        def div_row(s, carry, t=t):
            cv = cnt[pl.ds(t * W + s, 16)]
            inv = (jnp.ones((16,), jnp.float32) / jnp.maximum(cv, 1.0))[0]
            for j in range(8):
                xbuf[0, s, pl.ds(j * 16, 16)] = (
                    xbuf[0, s, pl.ds(j * 16, 16)] * inv)
            return carry
        lax.fori_loop(0, W, div_row, 0)
        pltpu.sync_copy(blk, out_hbm.at[pl.ds(seg_lo + t * W, W)])


def kernel(x, batch):
    b32 = batch.astype(jnp.int32)
    edges = jnp.arange(NW + 1, dtype=jnp.int32) * SPW
    bounds = jnp.searchsorted(b32, edges, side="left").astype(jnp.int32)
    bounds = jnp.concatenate([bounds, jnp.zeros((15,), jnp.int32)])

    mesh = plsc.VectorSubcoreMesh(core_axis_name="c", subcore_axis_name="s")
    out = pl.kernel(
        _pool_kernel,
        mesh=mesh,
        compiler_params=pltpu.CompilerParams(needs_layout_passes=False),
        out_type=jax.ShapeDtypeStruct((S_PAD, D), jnp.float32),
        scratch_types=[
            pltpu.VMEM((NB, C, D), jnp.float32),
            pltpu.VMEM((BCH,), jnp.int32),
            pltpu.VMEM((NB, C), jnp.int32),
            pltpu.VMEM((CNT_PAD,), jnp.float32),
            pltpu.VMEM((48,), jnp.int32),
            pltpu.VMEM_SHARED((16 * RPW, D), jnp.float32),
            pltpu.SemaphoreType.DMA,
            pltpu.SemaphoreType.DMA,
            pltpu.SemaphoreType.DMA,
            pltpu.SemaphoreType.DMA,
            pltpu.SemaphoreType.DMA,
            pltpu.SemaphoreType.DMA,
            pltpu.SemaphoreType.DMA,
            pltpu.SemaphoreType.DMA,
        ],
    )(x, b32, bounds)
    return out[:S]


# speedup vs baseline: 2.7440x; 1.2888x over previous
import jax
import jax.numpy as jnp
from jax import lax
from jax.experimental import pallas as pl
from jax.experimental.pallas import tpu as pltpu
from jax.experimental.pallas import tpu_sc as plsc

N = 320000
D = 128
S = 10000
NW = 32
SPW = 320
S_PAD = NW * SPW
C = 128
SUB = 16
BCH = SUB * C
RPW = SPW + 8
CNT_PAD = ((SPW + 15) // 16 + 1) * 16


NB = 4


def _pool_kernel(x_hbm, b_hbm, bounds_hbm, out_hbm,
                 xbuf, bbuf, idxb, cnt, bnds, acc_sh,
                 sx0, sx1, sx2, sx3, ss0, ss1, ss2, ss3):
    cid = lax.axis_index("c")
    sid = lax.axis_index("s")
    w = sid * 2 + cid
    seg_lo = w * SPW
    base = sid * RPW

    sx = (sx0, sx1, sx2, sx3)
    ss = (ss0, ss1, ss2, ss3)

    pltpu.sync_copy(bounds_hbm, bnds)
    bv0 = bnds[pl.ds(w, 16)]
    lo = bv0[0]
    hi = bv0[1]

    zeros16 = jnp.zeros((16,), jnp.float32)

    def zrow(i, carry):
        for j in range(8):
            xbuf[0, i, pl.ds(j * 16, 16)] = zeros16
        return carry
    lax.fori_loop(0, C, zrow, 0)
    pltpu.sync_copy(xbuf.at[0], acc_sh.at[pl.ds(base, C)])
    pltpu.sync_copy(xbuf.at[0], acc_sh.at[pl.ds(base + C, C)])
    pltpu.sync_copy(xbuf.at[0].at[pl.ds(0, RPW - 2 * C)],
                    acc_sh.at[pl.ds(base + 2 * C, RPW - 2 * C)])
    for j in range(CNT_PAD // 16):
        cnt[pl.ds(j * 16, 16)] = zeros16

    lo_al = lo & jnp.int32(~7)
    nchunks = (hi - lo_al + C - 1) // C

    def xload(k, b):
        r_eff = pl.multiple_of(jnp.minimum(lo_al + k * C, N - C), 8)
        pltpu.async_copy(x_hbm.at[pl.ds(r_eff, C)], xbuf.at[b], sx[b])

    def xwait(b):
        pltpu.make_async_copy(x_hbm.at[pl.ds(0, C)], xbuf.at[b],
                              sx[b]).wait()

    def scat_wait(b):
        pltpu.make_async_copy(xbuf.at[b], acc_sh.at[idxb.at[b]],
                              ss[b]).wait()

    @pl.when(nchunks > 0)
    def _():
        xload(0, 0)

    def pair(p, carry):
        for b in range(NB):
            k = NB * p + b

            @pl.when(k < nchunks)
            def _():
                s_sup = k // SUB
                rb_eff = pl.multiple_of(
                    jnp.minimum(lo_al + s_sup * BCH, N - BCH), 8)

                if b == 0:
                    @pl.when(lax.rem(k, SUB) == 0)
                    def _():
                        pltpu.sync_copy(b_hbm.at[pl.ds(rb_eff, BCH)], bbuf)

                @pl.when(k >= NB - 1)
                def _():
                    scat_wait((b + 1) % NB)

                @pl.when(k + 1 < nchunks)
                def _():
                    xload(k + 1, (b + 1) % NB)

                r = lo_al + k * C
                r_eff = pl.multiple_of(jnp.minimum(r, N - C), 8)
                off = r_eff - rb_eff
                vlo = jnp.maximum(r, lo)
                for j in range(C // 16):
                    bv = bbuf[pl.ds(off + j * 16, 16)]
                    g = r_eff + j * 16 + lax.iota(jnp.int32, 16)
                    valid = (g >= vlo) & (g < hi)
                    loc = jnp.where(valid, bv - seg_lo, SPW)
                    idxb[b, pl.ds(j * 16, 16)] = base + loc
                    ones = jnp.where(valid, 1.0, 0.0).astype(jnp.float32)
                    plsc.addupdate_scatter(cnt, [loc], ones)

                xwait(b)
                pltpu.async_copy(xbuf.at[b], acc_sh.at[idxb.at[b]], ss[b],
                                 add=True)
        return carry
    lax.fori_loop(0, (nchunks + NB - 1) // NB, pair, 0)

    for b in range(NB):
        cond = jnp.bool_(False)
        for t in range(1, NB):
            cond = cond | ((nchunks >= t) & (lax.rem(nchunks - t, NB) == b))

        @pl.when(cond)
        def _(b=b):
            scat_wait(b)

    W = 64
    blk = xbuf.at[0].at[pl.ds(0, W)]
    for t in range(SPW // W):
        pltpu.sync_copy(acc_sh.at[pl.ds(base + t * W, W)], blk)

        def div_row(s, carry, t=t):
            cv = cnt[pl.ds(t * W + s, 16)]
            inv = (jnp.ones((16,), jnp.float32) / jnp.maximum(cv, 1.0))[0]
            for j in range(8):
                xbuf[0, s, pl.ds(j * 16, 16)] = (
                    xbuf[0, s, pl.ds(j * 16, 16)] * inv)
            return carry
        lax.fori_loop(0, W, div_row, 0)
        pltpu.sync_copy(blk, out_hbm.at[pl.ds(seg_lo + t * W, W)])


def kernel(x, batch):
    b32 = batch.astype(jnp.int32)
    edges = jnp.arange(48, dtype=jnp.int32) * SPW
    bounds = jnp.sum((b32[:, None] < edges[None, :]).astype(jnp.int32),
                     axis=0, dtype=jnp.int32)

    mesh = plsc.VectorSubcoreMesh(core_axis_name="c", subcore_axis_name="s")
    out = pl.kernel(
        _pool_kernel,
        mesh=mesh,
        compiler_params=pltpu.CompilerParams(needs_layout_passes=False),
        out_type=jax.ShapeDtypeStruct((S_PAD, D), jnp.float32),
        scratch_types=[
            pltpu.VMEM((NB, C, D), jnp.float32),
            pltpu.VMEM((BCH,), jnp.int32),
            pltpu.VMEM((NB, C), jnp.int32),
            pltpu.VMEM((CNT_PAD,), jnp.float32),
            pltpu.VMEM((48,), jnp.int32),
            pltpu.VMEM_SHARED((16 * RPW, D), jnp.float32),
            pltpu.SemaphoreType.DMA,
            pltpu.SemaphoreType.DMA,
            pltpu.SemaphoreType.DMA,
            pltpu.SemaphoreType.DMA,
            pltpu.SemaphoreType.DMA,
            pltpu.SemaphoreType.DMA,
            pltpu.SemaphoreType.DMA,
            pltpu.SemaphoreType.DMA,
        ],
    )(x, b32, bounds)
    return out[:S]
